# Initial kernel scaffold; baseline (speedup 1.0000x reference)
import jax
import jax.numpy as jnp
from jax.experimental import pallas as pl


def kernel(center_traj, traj_x, traj_edge_index, traj_freq, geo_x, geo_edge_index, geo_freq, ptr, batch, traj_W, traj_b, geo_W, geo_b, Wqkv, bqkv, Wo, bo, Wl, bl):
    raise NotImplementedError("write your pallas kernel here")


# speedup vs baseline: 8.7603x; 8.7603x over previous
import functools

import jax
import jax.numpy as jnp
from jax import lax
from jax.experimental import pallas as pl
from jax.experimental.pallas import tpu as pltpu
from jax.experimental.pallas import tpu_sc as plsc

NC = 2
NS = 16
NW = NC * NS
CH = 80
RB = 2000

N = 10000
E = 320000
D = 128
B = 50
PER = 200
GPB = 10
NBLK = N // RB
NCHUNK = E // (NW * CH)
ST = 624
ST_LAST = N - (NS - 1) * ST


def _striped(copyfn):
    s = lax.axis_index("s")

    @pl.when(s < NS - 1)
    def _():
        copyfn(s * ST, ST)

    @pl.when(s == NS - 1)
    def _():
        copyfn((NS - 1) * ST, ST_LAST)


def _mesh():
    return plsc.VectorSubcoreMesh(
        core_axis_name="c", subcore_axis_name="s", num_cores=NC,
        num_subcores=NS)


def _deg_body(dstT_hbm, dstG_hbm, ones_hbm, degT_hbm, degG_hbm,
              dTv, dGv, ones_v, accT, accG):
    c = lax.axis_index("c")
    s = lax.axis_index("s")
    wid = s * NC + c
    ew = E // NW

    def init(start, size):
        sl = pl.ds(start, size)
        pltpu.sync_copy(ones_hbm.at[sl], accT.at[sl])
        pltpu.sync_copy(ones_hbm.at[sl], accG.at[sl])

    _striped(init)
    pltpu.sync_copy(ones_hbm.at[pl.ds(0, CH)], ones_v)
    plsc.subcore_barrier()

    def body(j, carry):
        base = wid * ew + j * CH
        pltpu.sync_copy(dstT_hbm.at[pl.ds(base, CH)], dTv)
        pltpu.sync_copy(ones_v, accT.at[dTv], add=True)
        pltpu.sync_copy(dstG_hbm.at[pl.ds(base, CH)], dGv)
        pltpu.sync_copy(ones_v, accG.at[dGv], add=True)
        return carry

    lax.fori_loop(0, NCHUNK, body, 0)
    plsc.subcore_barrier()

    def wb(start, size):
        sl = pl.ds(start, size)
        pltpu.sync_copy(accT.at[sl], degT_hbm.at[c, sl])
        pltpu.sync_copy(accG.at[sl], degG_hbm.at[c, sl])

    _striped(wb)


def _deg_call(dstT3, dstG3, ones16):
    f = pl.kernel(
        _deg_body,
        out_type=(jax.ShapeDtypeStruct((NC, N, 16), jnp.float32),
                  jax.ShapeDtypeStruct((NC, N, 16), jnp.float32)),
        mesh=_mesh(),
        scratch_types=[
            pltpu.VMEM((CH,), jnp.int32),
            pltpu.VMEM((CH,), jnp.int32),
            pltpu.VMEM((CH, 16), jnp.float32),
            pltpu.VMEM_SHARED((N, 16), jnp.float32),
            pltpu.VMEM_SHARED((N, 16), jnp.float32),
        ],
    )
    return f(dstT3, dstG3, ones16)


def _scatter_body(h_hbm, z_hbm, src_hbm, dst_hbm, p_hbm,
                  srcv, dstv, rows, acc, gsem):
    c = lax.axis_index("c")
    s = lax.axis_index("s")
    wid = s * NC + c

    def init(start, size):
        sl = pl.ds(start, size)

        @pl.when(c == 0)
        def _():
            pltpu.sync_copy(h_hbm.at[sl], acc.at[sl])

        @pl.when(c == 1)
        def _():
            pltpu.sync_copy(z_hbm.at[sl], acc.at[sl])

    _striped(init)
    plsc.subcore_barrier()
    ew = E // NW

    def body(j, carry):
        base = wid * ew + j * CH
        pltpu.sync_copy(src_hbm.at[pl.ds(base, CH)], srcv)
        pltpu.async_copy(h_hbm.at[srcv], rows, gsem).wait()
        pltpu.sync_copy(dst_hbm.at[pl.ds(base, CH)], dstv)
        pltpu.sync_copy(rows, acc.at[dstv], add=True)
        return carry

    lax.fori_loop(0, NCHUNK, body, 0)
    plsc.subcore_barrier()
    _striped(lambda start, size: pltpu.sync_copy(
        acc.at[pl.ds(start, size)], p_hbm.at[c, pl.ds(start, size)]))


def _scatter_call(h, zN, src3, dst3):
    f = pl.kernel(
        _scatter_body,
        out_type=jax.ShapeDtypeStruct((NC, N, D), jnp.float32),
        mesh=_mesh(),
        scratch_types=[
            pltpu.VMEM((CH,), jnp.int32),
            pltpu.VMEM((CH,), jnp.int32),
            pltpu.VMEM((CH, D), jnp.float32),
            pltpu.VMEM_SHARED((N, D), jnp.float32),
            pltpu.SemaphoreType.DMA,
        ],
    )
    return f(h, zN, src3, dst3)


def _dinv_body(degT_ref, degG_ref, dT_ref, dG_ref):
    sT = degT_ref[0] + degT_ref[1]
    sG = degG_ref[0] + degG_ref[1]
    dT = lax.rsqrt(sT[:, 0:1] - 1.0)
    dG = lax.rsqrt(sG[:, 0:1] - 1.0)
    dT_ref[...] = jnp.broadcast_to(dT, (RB, D))
    dG_ref[...] = jnp.broadcast_to(dG, (RB, D))


def _dinv_call(degT, degG):
    return pl.pallas_call(
        _dinv_body,
        grid=(NBLK,),
        in_specs=[
            pl.BlockSpec((NC, RB, 16), lambda i: (0, i, 0)),
            pl.BlockSpec((NC, RB, 16), lambda i: (0, i, 0)),
        ],
        out_specs=[
            pl.BlockSpec((RB, D), lambda i: (i, 0)),
            pl.BlockSpec((RB, D), lambda i: (i, 0)),
        ],
        out_shape=[
            jax.ShapeDtypeStruct((N, D), jnp.float32),
            jax.ShapeDtypeStruct((N, D), jnp.float32),
        ],
    )(degT, degG)


def _mm_body(x_ref, w_ref, dinv_ref, h_ref):
    h = jnp.dot(x_ref[...], w_ref[...], preferred_element_type=jnp.float32)
    h_ref[...] = h * dinv_ref[...]


def _mm_call(x, W, dinvb):
    return pl.pallas_call(
        _mm_body,
        grid=(NBLK,),
        in_specs=[
            pl.BlockSpec((RB, D), lambda i: (i, 0)),
            pl.BlockSpec((D, D), lambda i: (0, 0)),
            pl.BlockSpec((RB, D), lambda i: (i, 0)),
        ],
        out_specs=pl.BlockSpec((RB, D), lambda i: (i, 0)),
        out_shape=jax.ShapeDtypeStruct((N, D), jnp.float32),
    )(x, W, dinvb)


def _comb_mm_body(p_ref, dinv_ref, b_ref, w_ref, h_ref):
    t = (p_ref[0] + p_ref[1]) * dinv_ref[...] + b_ref[...]
    h = jnp.dot(t, w_ref[...], preferred_element_type=jnp.float32)
    h_ref[...] = h * dinv_ref[...]


def _comb_mm_call(p, dinvb, b2, W):
    return pl.pallas_call(
        _comb_mm_body,
        grid=(NBLK,),
        in_specs=[
            pl.BlockSpec((NC, RB, D), lambda i: (0, i, 0)),
            pl.BlockSpec((RB, D), lambda i: (i, 0)),
            pl.BlockSpec((1, D), lambda i: (0, 0)),
            pl.BlockSpec((D, D), lambda i: (0, 0)),
        ],
        out_specs=pl.BlockSpec((RB, D), lambda i: (i, 0)),
        out_shape=jax.ShapeDtypeStruct((N, D), jnp.float32),
    )(p, dinvb, b2, W)


def _pool(t, freq_row):
    kept = (freq_row >= 0.5).astype(jnp.float32)
    seg = lax.broadcasted_iota(jnp.int32, (16, RB), 1) // PER
    gid = lax.broadcasted_iota(jnp.int32, (16, RB), 0)
    K = jnp.where(seg == gid, 1.0, 0.0) * kept
    ssum = jnp.dot(K, t, preferred_element_type=jnp.float32)
    cnt = jnp.sum(K, axis=1, keepdims=True)
    return ssum / jnp.maximum(cnt, 1.0)


def _final_body(p_ref, dinv_ref, b_ref, freq_ref, pool_ref):
    t = (p_ref[0] + p_ref[1]) * dinv_ref[...] + b_ref[...]
    fr = freq_ref[...].reshape(1, RB)
    pool_ref[...] = _pool(t, fr).reshape(1, 1, 16, D)


def _final_call(p, dinvb, b2, freq3):
    return pl.pallas_call(
        _final_body,
        grid=(NBLK,),
        in_specs=[
            pl.BlockSpec((NC, RB, D), lambda i: (0, i, 0)),
            pl.BlockSpec((RB, D), lambda i: (i, 0)),
            pl.BlockSpec((1, D), lambda i: (0, 0)),
            pl.BlockSpec((1, 1, RB), lambda i: (i, 0, 0)),
        ],
        out_specs=pl.BlockSpec((1, 1, 16, D), lambda i: (i, 0, 0, 0)),
        out_shape=jax.ShapeDtypeStruct((NBLK, 1, 16, D), jnp.float32),
    )(p, dinvb, b2, freq3)


def _final_center_body(p_ref, dinv_ref, b_ref, freq_ref, ctr_ref,
                       pool_ref, ctro_ref):
    t = (p_ref[0] + p_ref[1]) * dinv_ref[...] + b_ref[...]
    fr = freq_ref[...].reshape(1, RB)
    pool_ref[...] = _pool(t, fr).reshape(1, 1, 16, D)
    idxl = ctr_ref[...].reshape(1, GPB * 20)
    rowid = lax.broadcasted_iota(jnp.int32, (RB, GPB * 20), 0)
    selT = (rowid == idxl).astype(jnp.float32)
    ctr = lax.dot_general(selT, t, (((0,), (0,)), ((), ())),
                          preferred_element_type=jnp.float32)
    ctro_ref[...] = ctr.reshape(1, 1, GPB * 20, D)


def _final_center_call(p, dinvb, b2, freq3, ctr3):
    return pl.pallas_call(
        _final_center_body,
        grid=(NBLK,),
        in_specs=[
            pl.BlockSpec((NC, RB, D), lambda i: (0, i, 0)),
            pl.BlockSpec((RB, D), lambda i: (i, 0)),
            pl.BlockSpec((1, D), lambda i: (0, 0)),
            pl.BlockSpec((1, 1, RB), lambda i: (i, 0, 0)),
            pl.BlockSpec((1, 1, GPB * 20), lambda i: (i, 0, 0)),
        ],
        out_specs=[
            pl.BlockSpec((1, 1, 16, D), lambda i: (i, 0, 0, 0)),
            pl.BlockSpec((1, 1, GPB * 20, D), lambda i: (i, 0, 0, 0)),
        ],
        out_shape=[
            jax.ShapeDtypeStruct((NBLK, 1, 16, D), jnp.float32),
            jax.ShapeDtypeStruct((NBLK, 1, GPB * 20, D), jnp.float32),
        ],
    )(p, dinvb, b2, freq3, ctr3)


def _attn_body(tp_ref, gp_ref, wv_ref, bv_ref, wo_ref, bo_ref,
               wl_ref, bl_ref, out_ref):
    per = jnp.concatenate([tp_ref[...], gp_ref[...]], axis=1)
    nt = (((1,), (1,)), ((), ()))
    v = lax.dot_general(per, wv_ref[...], nt,
                        preferred_element_type=jnp.float32) + bv_ref[...]
    o = lax.dot_general(v, wo_ref[...], nt,
                        preferred_element_type=jnp.float32) + bo_ref[...]
    up = lax.dot_general(o, wl_ref[...], nt,
                         preferred_element_type=jnp.float32) + bl_ref[...]
    out_ref[...] = up


def _attn_call(tp, gp, Wv, bv, Wo, bo, Wl, bl):
    e2 = 2 * D
    full = lambda shape: pl.BlockSpec(shape, lambda: tuple(0 for _ in shape))
    return pl.pallas_call(
        _attn_body,
        in_specs=[
            full((B, D)), full((B, D)),
            full((e2, e2)), full((1, e2)),
            full((e2, e2)), full((1, e2)),
            full((D, e2)), full((1, D)),
        ],
        out_specs=full((B, D)),
        out_shape=jax.ShapeDtypeStruct((B, D), jnp.float32),
    )(tp, gp, Wv, bv, Wo, bo, Wl, bl)


def _stack(x, src3, dst3, dinvb, Ws, bs, zN):
    h = _mm_call(x, Ws[0], dinvb)
    p = _scatter_call(h, zN, src3, dst3)
    h = _comb_mm_call(p, dinvb, bs[0:1], Ws[1])
    p = _scatter_call(h, zN, src3, dst3)
    h = _comb_mm_call(p, dinvb, bs[1:2], Ws[2])
    p = _scatter_call(h, zN, src3, dst3)
    return p


def kernel(center_traj, traj_x, traj_edge_index, traj_freq, geo_x,
           geo_edge_index, geo_freq, ptr, batch, traj_W, traj_b, geo_W,
           geo_b, Wqkv, bqkv, Wo, bo, Wl, bl):
    srcT3 = traj_edge_index[0]
    dstT3 = traj_edge_index[1]
    srcG3 = geo_edge_index[0]
    dstG3 = geo_edge_index[1]
    ones16 = jnp.ones((N, 16), jnp.float32)
    zN = jnp.zeros((N, D), jnp.float32)

    degT, degG = _deg_call(dstT3, dstG3, ones16)
    dinvT, dinvG = _dinv_call(degT, degG)

    pT = _stack(traj_x, srcT3, dstT3, dinvT, traj_W, traj_b, zN)
    pG = _stack(geo_x, srcG3, dstG3, dinvG, geo_W, geo_b, zN)

    freqT3 = traj_freq.reshape(NBLK, 1, RB)
    freqG3 = geo_freq.reshape(NBLK, 1, RB)
    gidx = jnp.arange(B, dtype=jnp.int32)
    local = center_traj + (ptr[:-1] - (gidx // GPB) * RB)[:, None]
    ctr3 = local.reshape(NBLK, 1, GPB * 20).astype(jnp.int32)

    poolT, ctrE = _final_center_call(pT, dinvT, traj_b[2:3], freqT3, ctr3)
    poolG = _final_call(pG, dinvG, geo_b[2:3], freqG3)

    tp = poolT.reshape(NBLK, 16, D)[:, :GPB].reshape(B, D)
    gp = poolG.reshape(NBLK, 16, D)[:, :GPB].reshape(B, D)

    e2 = 2 * D
    Wv = Wqkv[2 * e2:]
    bv = bqkv[2 * e2:].reshape(1, e2)
    up = _attn_call(tp, gp, Wv, bv, Wo, bo.reshape(1, e2),
                    Wl, bl.reshape(1, D))

    center_traj_emb = ctrE.reshape(B, 20, D)
    user_perfence = up.reshape(B, 1, D)
    return (center_traj_emb, user_perfence)


# speedup vs baseline: 17.0753x; 1.9492x over previous
import functools

import jax
import jax.numpy as jnp
from jax import lax
from jax.experimental import pallas as pl
from jax.experimental.pallas import tpu as pltpu
from jax.experimental.pallas import tpu_sc as plsc

NC = 2
NS = 16
CH = 80
RB = 2000

N = 10000
E = 320000
D = 128
B = 50
PER = 200
GPB = 10
NBLK = N // RB
EW = E // NS
NCHUNK = EW // CH
NPAIR = NCHUNK // 2
ST = 624
ST_LAST = N - (NS - 1) * ST


def _striped(copyfn):
    s = lax.axis_index("s")

    @pl.when(s < NS - 1)
    def _():
        copyfn(s * ST, ST)

    @pl.when(s == NS - 1)
    def _():
        copyfn((NS - 1) * ST, ST_LAST)


def _mesh():
    return plsc.VectorSubcoreMesh(
        core_axis_name="c", subcore_axis_name="s", num_cores=NC,
        num_subcores=NS)


def _deg_core(dst_hbm, ones_hbm, deg_hbm, d0, d1, ones_v, acc, sem0, sem1):
    s = lax.axis_index("s")
    base0 = s * EW

    def init(start, size):
        sl = pl.ds(start, size)
        pltpu.sync_copy(ones_hbm.at[sl], acc.at[sl])

    _striped(init)
    pltpu.sync_copy(ones_hbm.at[pl.ds(0, CH)], ones_v)
    plsc.subcore_barrier()

    pltpu.sync_copy(dst_hbm.at[pl.ds(base0, CH)], d0)
    pltpu.async_copy(dst_hbm.at[pl.ds(base0 + CH, CH)], d1, sem1)

    def body(i, carry):
        base = base0 + i * 2 * CH
        pltpu.sync_copy(ones_v, acc.at[d0], add=True)

        @pl.when(i < NPAIR - 1)
        def _():
            pltpu.async_copy(dst_hbm.at[pl.ds(base + 2 * CH, CH)], d0, sem0)

        pltpu.make_async_copy(dst_hbm.at[pl.ds(0, CH)], d1, sem1).wait()
        pltpu.sync_copy(ones_v, acc.at[d1], add=True)

        @pl.when(i < NPAIR - 1)
        def _():
            pltpu.async_copy(dst_hbm.at[pl.ds(base + 3 * CH, CH)], d1, sem1)
            pltpu.make_async_copy(dst_hbm.at[pl.ds(0, CH)], d0, sem0).wait()

        return carry

    lax.fori_loop(0, NPAIR, body, 0)
    plsc.subcore_barrier()
    _striped(lambda start, size: pltpu.sync_copy(
        acc.at[pl.ds(start, size)], deg_hbm.at[pl.ds(start, size)]))


def _deg_body(dstT_hbm, dstG_hbm, ones_hbm, degT_hbm, degG_hbm,
              d0, d1, ones_v, acc, sem0, sem1):
    c = lax.axis_index("c")

    @pl.when(c == 0)
    def _():
        _deg_core(dstT_hbm, ones_hbm, degT_hbm, d0, d1, ones_v, acc,
                  sem0, sem1)

    @pl.when(c == 1)
    def _():
        _deg_core(dstG_hbm, ones_hbm, degG_hbm, d0, d1, ones_v, acc,
                  sem0, sem1)


def _deg_call(dstT, dstG, ones16):
    f = pl.kernel(
        _deg_body,
        out_type=(jax.ShapeDtypeStruct((N, 16), jnp.float32),
                  jax.ShapeDtypeStruct((N, 16), jnp.float32)),
        mesh=_mesh(),
        scratch_types=[
            pltpu.VMEM((CH,), jnp.int32),
            pltpu.VMEM((CH,), jnp.int32),
            pltpu.VMEM((CH, 16), jnp.float32),
            pltpu.VMEM_SHARED((N, 16), jnp.float32),
            pltpu.SemaphoreType.DMA,
            pltpu.SemaphoreType.DMA,
        ],
    )
    return f(dstT, dstG, ones16)


def _edge_core(h_hbm, src_hbm, dst_hbm, p_hbm,
               s0, d0, s1, d1, rows0, rows1, acc,
               isem0, isem1, gsem0, gsem1):
    s = lax.axis_index("s")
    base0 = s * EW

    def init(start, size):
        sl = pl.ds(start, size)
        pltpu.sync_copy(h_hbm.at[sl], acc.at[sl])

    _striped(init)
    plsc.subcore_barrier()

    pltpu.sync_copy(src_hbm.at[pl.ds(base0, CH)], s0)
    pltpu.sync_copy(dst_hbm.at[pl.ds(base0, CH)], d0)
    pltpu.async_copy(h_hbm.at[s0], rows0, gsem0)
    pltpu.async_copy(src_hbm.at[pl.ds(base0 + CH, CH)], s1, isem1)
    pltpu.async_copy(dst_hbm.at[pl.ds(base0 + CH, CH)], d1, isem1)

    def body(i, carry):
        base = base0 + i * 2 * CH
        pltpu.make_async_copy(src_hbm.at[pl.ds(0, CH)], s1, isem1).wait()
        pltpu.make_async_copy(dst_hbm.at[pl.ds(0, CH)], d1, isem1).wait()
        pltpu.make_async_copy(h_hbm.at[s0], rows0, gsem0).wait()
        pltpu.async_copy(h_hbm.at[s1], rows1, gsem1)
        pltpu.sync_copy(rows0, acc.at[d0], add=True)

        @pl.when(i < NPAIR - 1)
        def _():
            pltpu.async_copy(src_hbm.at[pl.ds(base + 2 * CH, CH)], s0, isem0)
            pltpu.async_copy(dst_hbm.at[pl.ds(base + 2 * CH, CH)], d0, isem0)

        pltpu.make_async_copy(h_hbm.at[s1], rows1, gsem1).wait()

        @pl.when(i < NPAIR - 1)
        def _():
            pltpu.make_async_copy(src_hbm.at[pl.ds(0, CH)], s0, isem0).wait()
            pltpu.make_async_copy(dst_hbm.at[pl.ds(0, CH)], d0, isem0).wait()
            pltpu.async_copy(h_hbm.at[s0], rows0, gsem0)
            pltpu.async_copy(src_hbm.at[pl.ds(base + 3 * CH, CH)], s1, isem1)
            pltpu.async_copy(dst_hbm.at[pl.ds(base + 3 * CH, CH)], d1, isem1)

        pltpu.sync_copy(rows1, acc.at[d1], add=True)
        return carry

    lax.fori_loop(0, NPAIR, body, 0)
    plsc.subcore_barrier()
    _striped(lambda start, size: pltpu.sync_copy(
        acc.at[pl.ds(start, size)], p_hbm.at[pl.ds(start, size)]))


def _edge_body(hT_hbm, hG_hbm, sT_hbm, dT_hbm, sG_hbm, dG_hbm,
               pT_hbm, pG_hbm, *scr):
    c = lax.axis_index("c")

    @pl.when(c == 0)
    def _():
        _edge_core(hT_hbm, sT_hbm, dT_hbm, pT_hbm, *scr)

    @pl.when(c == 1)
    def _():
        _edge_core(hG_hbm, sG_hbm, dG_hbm, pG_hbm, *scr)


def _edge_call(hT, hG, sT, dT, sG, dG):
    f = pl.kernel(
        _edge_body,
        out_type=(jax.ShapeDtypeStruct((N, D), jnp.float32),
                  jax.ShapeDtypeStruct((N, D), jnp.float32)),
        mesh=_mesh(),
        scratch_types=[
            pltpu.VMEM((CH,), jnp.int32),
            pltpu.VMEM((CH,), jnp.int32),
            pltpu.VMEM((CH,), jnp.int32),
            pltpu.VMEM((CH,), jnp.int32),
            pltpu.VMEM((CH, D), jnp.float32),
            pltpu.VMEM((CH, D), jnp.float32),
            pltpu.VMEM_SHARED((N, D), jnp.float32),
            pltpu.SemaphoreType.DMA,
            pltpu.SemaphoreType.DMA,
            pltpu.SemaphoreType.DMA,
            pltpu.SemaphoreType.DMA,
        ],
    )
    return f(hT, hG, sT, dT, sG, dG)


def _dinv_body(degT_ref, degG_ref, dT_ref, dG_ref):
    dT = lax.rsqrt(degT_ref[...][:, 0:1])
    dG = lax.rsqrt(degG_ref[...][:, 0:1])
    dT_ref[...] = jnp.broadcast_to(dT, (RB, D))
    dG_ref[...] = jnp.broadcast_to(dG, (RB, D))


def _dinv_call(degT, degG):
    return pl.pallas_call(
        _dinv_body,
        grid=(NBLK,),
        in_specs=[
            pl.BlockSpec((RB, 16), lambda i: (i, 0)),
            pl.BlockSpec((RB, 16), lambda i: (i, 0)),
        ],
        out_specs=[
            pl.BlockSpec((RB, D), lambda i: (i, 0)),
            pl.BlockSpec((RB, D), lambda i: (i, 0)),
        ],
        out_shape=[
            jax.ShapeDtypeStruct((N, D), jnp.float32),
            jax.ShapeDtypeStruct((N, D), jnp.float32),
        ],
    )(degT, degG)


def _mm_body(x_ref, w_ref, dinv_ref, h_ref):
    h = jnp.dot(x_ref[...], w_ref[...], preferred_element_type=jnp.float32)
    h_ref[...] = h * dinv_ref[...]


def _mm_call(x, W, dinvb):
    return pl.pallas_call(
        _mm_body,
        grid=(NBLK,),
        in_specs=[
            pl.BlockSpec((RB, D), lambda i: (i, 0)),
            pl.BlockSpec((D, D), lambda i: (0, 0)),
            pl.BlockSpec((RB, D), lambda i: (i, 0)),
        ],
        out_specs=pl.BlockSpec((RB, D), lambda i: (i, 0)),
        out_shape=jax.ShapeDtypeStruct((N, D), jnp.float32),
    )(x, W, dinvb)


def _comb_mm_body(p_ref, dinv_ref, b_ref, w_ref, h_ref):
    t = p_ref[...] * dinv_ref[...] + b_ref[...]
    h = jnp.dot(t, w_ref[...], preferred_element_type=jnp.float32)
    h_ref[...] = h * dinv_ref[...]


def _comb_mm_call(p, dinvb, b2, W):
    return pl.pallas_call(
        _comb_mm_body,
        grid=(NBLK,),
        in_specs=[
            pl.BlockSpec((RB, D), lambda i: (i, 0)),
            pl.BlockSpec((RB, D), lambda i: (i, 0)),
            pl.BlockSpec((1, D), lambda i: (0, 0)),
            pl.BlockSpec((D, D), lambda i: (0, 0)),
        ],
        out_specs=pl.BlockSpec((RB, D), lambda i: (i, 0)),
        out_shape=jax.ShapeDtypeStruct((N, D), jnp.float32),
    )(p, dinvb, b2, W)


def _pool(t, freq_row):
    kept = (freq_row >= 0.5).astype(jnp.float32)
    seg = lax.broadcasted_iota(jnp.int32, (16, RB), 1) // PER
    gid = lax.broadcasted_iota(jnp.int32, (16, RB), 0)
    K = jnp.where(seg == gid, 1.0, 0.0) * kept
    ssum = jnp.dot(K, t, preferred_element_type=jnp.float32)
    cnt = jnp.sum(K, axis=1, keepdims=True)
    return ssum / jnp.maximum(cnt, 1.0)


def _final_body(p_ref, dinv_ref, b_ref, freq_ref, pool_ref):
    t = p_ref[...] * dinv_ref[...] + b_ref[...]
    fr = freq_ref[...].reshape(1, RB)
    pool_ref[...] = _pool(t, fr).reshape(1, 1, 16, D)


def _final_call(p, dinvb, b2, freq3):
    return pl.pallas_call(
        _final_body,
        grid=(NBLK,),
        in_specs=[
            pl.BlockSpec((RB, D), lambda i: (i, 0)),
            pl.BlockSpec((RB, D), lambda i: (i, 0)),
            pl.BlockSpec((1, D), lambda i: (0, 0)),
            pl.BlockSpec((1, 1, RB), lambda i: (i, 0, 0)),
        ],
        out_specs=pl.BlockSpec((1, 1, 16, D), lambda i: (i, 0, 0, 0)),
        out_shape=jax.ShapeDtypeStruct((NBLK, 1, 16, D), jnp.float32),
    )(p, dinvb, b2, freq3)


def _final_center_body(p_ref, dinv_ref, b_ref, freq_ref, ctr_ref,
                       pool_ref, ctro_ref):
    t = p_ref[...] * dinv_ref[...] + b_ref[...]
    fr = freq_ref[...].reshape(1, RB)
    pool_ref[...] = _pool(t, fr).reshape(1, 1, 16, D)
    idxl = ctr_ref[...].reshape(1, GPB * 20)
    rowid = lax.broadcasted_iota(jnp.int32, (RB, GPB * 20), 0)
    selT = (rowid == idxl).astype(jnp.float32)
    ctr = lax.dot_general(selT, t, (((0,), (0,)), ((), ())),
                          preferred_element_type=jnp.float32)
    ctro_ref[...] = ctr.reshape(1, 1, GPB * 20, D)


def _final_center_call(p, dinvb, b2, freq3, ctr3):
    return pl.pallas_call(
        _final_center_body,
        grid=(NBLK,),
        in_specs=[
            pl.BlockSpec((RB, D), lambda i: (i, 0)),
            pl.BlockSpec((RB, D), lambda i: (i, 0)),
            pl.BlockSpec((1, D), lambda i: (0, 0)),
            pl.BlockSpec((1, 1, RB), lambda i: (i, 0, 0)),
            pl.BlockSpec((1, 1, GPB * 20), lambda i: (i, 0, 0)),
        ],
        out_specs=[
            pl.BlockSpec((1, 1, 16, D), lambda i: (i, 0, 0, 0)),
            pl.BlockSpec((1, 1, GPB * 20, D), lambda i: (i, 0, 0, 0)),
        ],
        out_shape=[
            jax.ShapeDtypeStruct((NBLK, 1, 16, D), jnp.float32),
            jax.ShapeDtypeStruct((NBLK, 1, GPB * 20, D), jnp.float32),
        ],
    )(p, dinvb, b2, freq3, ctr3)


def _attn_body(tp_ref, gp_ref, wv_ref, bv_ref, wo_ref, bo_ref,
               wl_ref, bl_ref, out_ref):
    per = jnp.concatenate([tp_ref[...], gp_ref[...]], axis=1)
    nt = (((1,), (1,)), ((), ()))
    v = lax.dot_general(per, wv_ref[...], nt,
                        preferred_element_type=jnp.float32) + bv_ref[...]
    o = lax.dot_general(v, wo_ref[...], nt,
                        preferred_element_type=jnp.float32) + bo_ref[...]
    up = lax.dot_general(o, wl_ref[...], nt,
                         preferred_element_type=jnp.float32) + bl_ref[...]
    out_ref[...] = up


def _attn_call(tp, gp, Wv, bv, Wo, bo, Wl, bl):
    e2 = 2 * D
    full = lambda shape: pl.BlockSpec(shape, lambda: tuple(0 for _ in shape))
    return pl.pallas_call(
        _attn_body,
        in_specs=[
            full((B, D)), full((B, D)),
            full((e2, e2)), full((1, e2)),
            full((e2, e2)), full((1, e2)),
            full((D, e2)), full((1, D)),
        ],
        out_specs=full((B, D)),
        out_shape=jax.ShapeDtypeStruct((B, D), jnp.float32),
    )(tp, gp, Wv, bv, Wo, bo, Wl, bl)


def kernel(center_traj, traj_x, traj_edge_index, traj_freq, geo_x,
           geo_edge_index, geo_freq, ptr, batch, traj_W, traj_b, geo_W,
           geo_b, Wqkv, bqkv, Wo, bo, Wl, bl):
    srcT = traj_edge_index[0]
    dstT = traj_edge_index[1]
    srcG = geo_edge_index[0]
    dstG = geo_edge_index[1]
    ones16 = jnp.ones((N, 16), jnp.float32)

    degT, degG = _deg_call(dstT, dstG, ones16)
    dinvT, dinvG = _dinv_call(degT, degG)

    hT = _mm_call(traj_x, traj_W[0], dinvT)
    hG = _mm_call(geo_x, geo_W[0], dinvG)
    pT, pG = _edge_call(hT, hG, srcT, dstT, srcG, dstG)
    hT = _comb_mm_call(pT, dinvT, traj_b[0:1], traj_W[1])
    hG = _comb_mm_call(pG, dinvG, geo_b[0:1], geo_W[1])
    pT, pG = _edge_call(hT, hG, srcT, dstT, srcG, dstG)
    hT = _comb_mm_call(pT, dinvT, traj_b[1:2], traj_W[2])
    hG = _comb_mm_call(pG, dinvG, geo_b[1:2], geo_W[2])
    pT, pG = _edge_call(hT, hG, srcT, dstT, srcG, dstG)

    freqT3 = traj_freq.reshape(NBLK, 1, RB)
    freqG3 = geo_freq.reshape(NBLK, 1, RB)
    gidx = jnp.arange(B, dtype=jnp.int32)
    local = center_traj + (ptr[:-1] - (gidx // GPB) * RB)[:, None]
    ctr3 = local.reshape(NBLK, 1, GPB * 20).astype(jnp.int32)

    poolT, ctrE = _final_center_call(pT, dinvT, traj_b[2:3], freqT3, ctr3)
    poolG = _final_call(pG, dinvG, geo_b[2:3], freqG3)

    tp = poolT.reshape(NBLK, 16, D)[:, :GPB].reshape(B, D)
    gp = poolG.reshape(NBLK, 16, D)[:, :GPB].reshape(B, D)

    e2 = 2 * D
    Wv = Wqkv[2 * e2:]
    bv = bqkv[2 * e2:].reshape(1, e2)
    up = _attn_call(tp, gp, Wv, bv, Wo, bo.reshape(1, e2),
                    Wl, bl.reshape(1, D))

    center_traj_emb = ctrE.reshape(B, 20, D)
    user_perfence = up.reshape(B, 1, D)
    return (center_traj_emb, user_perfence)


# speedup vs baseline: 23.4054x; 1.3707x over previous
import functools

import jax
import jax.numpy as jnp
from jax import lax
from jax.experimental import pallas as pl
from jax.experimental.pallas import tpu as pltpu
from jax.experimental.pallas import tpu_sc as plsc

NC = 2
NS = 16
CH = 128

N = 10000
D = 128
E = 320000
B = 50
PER = 200
GPB = 10

N2 = 10240
NPAD = N2 - N
RB = 2048
NBLK = N2 // RB
PRB = 2000
NROW = E // CH
NROWP = 2560
WROW = NROWP // NS
QR = WROW // 4
STRIPE = N2 // NS


def _mesh():
    return plsc.VectorSubcoreMesh(
        core_axis_name="c", subcore_axis_name="s", num_cores=NC,
        num_subcores=NS)


def _deg_core(dst_hbm, ones_hbm, deg_hbm, dbuf, ones_v, acc, sem0, sem1):
    s = lax.axis_index("s")
    stripe = pl.ds(s * STRIPE, STRIPE)
    sems = (sem0, sem1)
    pltpu.sync_copy(ones_hbm.at[stripe], acc.at[stripe])
    pltpu.sync_copy(ones_hbm.at[pl.ds(0, CH)], ones_v)
    pltpu.sync_copy(dst_hbm.at[pl.ds(s * WROW, WROW)], dbuf)
    plsc.subcore_barrier()

    def scat(c, k):
        pltpu.async_copy(ones_v, acc.at[dbuf.at[c]], sems[k], add=True)

    def wait_scat(c, k):
        pltpu.make_async_copy(ones_v, acc.at[dbuf.at[c]], sems[k]).wait()

    scat(0, 0)
    scat(1, 1)

    def body(t, carry):
        for k in range(2):
            c = 2 + 2 * t + k
            wait_scat(c - 2, k)
            scat(c, k)
        return carry

    lax.fori_loop(0, (WROW - 2) // 2, body, 0)
    wait_scat(WROW - 2, 0)
    wait_scat(WROW - 1, 1)
    plsc.subcore_barrier()
    pltpu.sync_copy(acc.at[stripe], deg_hbm.at[stripe])


def _deg_body(dstT_hbm, dstG_hbm, ones_hbm, degT_hbm, degG_hbm,
              dbuf, ones_v, acc, sem0, sem1):
    c = lax.axis_index("c")

    @pl.when(c == 0)
    def _():
        _deg_core(dstT_hbm, ones_hbm, degT_hbm, dbuf, ones_v, acc,
                  sem0, sem1)

    @pl.when(c == 1)
    def _():
        _deg_core(dstG_hbm, ones_hbm, degG_hbm, dbuf, ones_v, acc,
                  sem0, sem1)


def _deg_call(dstT, dstG, ones16):
    f = pl.kernel(
        _deg_body,
        out_type=(jax.ShapeDtypeStruct((N2, 16), jnp.float32),
                  jax.ShapeDtypeStruct((N2, 16), jnp.float32)),
        mesh=_mesh(),
        scratch_types=[
            pltpu.VMEM((WROW, CH), jnp.int32),
            pltpu.VMEM((CH, 16), jnp.float32),
            pltpu.VMEM_SHARED((N2, 16), jnp.float32),
            pltpu.SemaphoreType.DMA,
            pltpu.SemaphoreType.DMA,
        ],
    )
    return f(dstT, dstG, ones16)


def _edge_core(h_hbm, src_hbm, dst_hbm, p_hbm,
               sbuf, dbuf, rows, acc, gsem, ssem):
    s = lax.axis_index("s")
    stripe = pl.ds(s * STRIPE, STRIPE)
    row0 = s * WROW
    pltpu.sync_copy(h_hbm.at[stripe], acc.at[stripe])
    plsc.subcore_barrier()

    def gath(c, k):
        pltpu.async_copy(h_hbm.at[sbuf.at[c]], rows[k], gsem[k])

    def wait_gath(c, k):
        pltpu.make_async_copy(h_hbm.at[sbuf.at[c]], rows[k], gsem[k]).wait()

    def scat(c, k):
        pltpu.async_copy(rows[k], acc.at[dbuf.at[c]], ssem[k], add=True)

    def wait_scat(c, k):
        pltpu.make_async_copy(rows[k], acc.at[dbuf.at[c]], ssem[k]).wait()

    for q in range(4):
        qrow = row0 + q * QR
        pltpu.sync_copy(src_hbm.at[pl.ds(qrow, QR)], sbuf)
        pltpu.sync_copy(dst_hbm.at[pl.ds(qrow, QR)], dbuf)
        gath(0, 0)
        gath(1, 1)
        wait_gath(0, 0)
        scat(0, 0)

        def body(t, carry):
            for k in range(2):
                c = 2 + 2 * t + k
                wait_scat(c - 2, k)
                gath(c, k)
                wait_gath(c - 1, 1 - k)
                scat(c - 1, 1 - k)
            return carry

        lax.fori_loop(0, (QR - 2) // 2, body, 0)
        wait_gath(QR - 1, 1)
        scat(QR - 1, 1)
        wait_scat(QR - 2, 0)
        wait_scat(QR - 1, 1)

    plsc.subcore_barrier()
    pltpu.sync_copy(acc.at[stripe], p_hbm.at[stripe])


def _edge_body(hT_hbm, hG_hbm, sT_hbm, dT_hbm, sG_hbm, dG_hbm,
               pT_hbm, pG_hbm, sbuf, dbuf, r0, r1, g0, g1, ss0, ss1, acc):
    c = lax.axis_index("c")
    rows = (r0, r1)
    gsem = (g0, g1)
    ssem = (ss0, ss1)

    @pl.when(c == 0)
    def _():
        _edge_core(hT_hbm, sT_hbm, dT_hbm, pT_hbm, sbuf, dbuf, rows, acc,
                   gsem, ssem)

    @pl.when(c == 1)
    def _():
        _edge_core(hG_hbm, sG_hbm, dG_hbm, pG_hbm, sbuf, dbuf, rows, acc,
                   gsem, ssem)


def _edge_call(hT, hG, sT, dT, sG, dG):
    f = pl.kernel(
        _edge_body,
        out_type=(jax.ShapeDtypeStruct((N2, D), jnp.float32),
                  jax.ShapeDtypeStruct((N2, D), jnp.float32)),
        mesh=_mesh(),
        scratch_types=[
            pltpu.VMEM((QR, CH), jnp.int32),
            pltpu.VMEM((QR, CH), jnp.int32),
            pltpu.VMEM((CH, D), jnp.float32),
            pltpu.VMEM((CH, D), jnp.float32),
            pltpu.SemaphoreType.DMA,
            pltpu.SemaphoreType.DMA,
            pltpu.SemaphoreType.DMA,
            pltpu.SemaphoreType.DMA,
            pltpu.VMEM_SHARED((N2, D), jnp.float32),
        ],
    )
    return f(hT, hG, sT, dT, sG, dG)


def _dinv_body(degT_ref, degG_ref, dT_ref, dG_ref):
    dT = lax.rsqrt(degT_ref[...][:, 0:1])
    dG = lax.rsqrt(degG_ref[...][:, 0:1])
    dT_ref[...] = jnp.broadcast_to(dT, (RB, D))
    dG_ref[...] = jnp.broadcast_to(dG, (RB, D))


def _dinv_call(degT, degG):
    return pl.pallas_call(
        _dinv_body,
        grid=(NBLK,),
        in_specs=[
            pl.BlockSpec((RB, 16), lambda i: (i, 0)),
            pl.BlockSpec((RB, 16), lambda i: (i, 0)),
        ],
        out_specs=[
            pl.BlockSpec((RB, D), lambda i: (i, 0)),
            pl.BlockSpec((RB, D), lambda i: (i, 0)),
        ],
        out_shape=[
            jax.ShapeDtypeStruct((N2, D), jnp.float32),
            jax.ShapeDtypeStruct((N2, D), jnp.float32),
        ],
    )(degT, degG)


def _mm_body(x_ref, w_ref, dinv_ref, h_ref):
    h = jnp.dot(x_ref[...], w_ref[...], preferred_element_type=jnp.float32)
    h_ref[...] = h * dinv_ref[...]


def _mm_call(x, W, dinvb):
    return pl.pallas_call(
        _mm_body,
        grid=(NBLK,),
        in_specs=[
            pl.BlockSpec((RB, D), lambda i: (i, 0)),
            pl.BlockSpec((D, D), lambda i: (0, 0)),
            pl.BlockSpec((RB, D), lambda i: (i, 0)),
        ],
        out_specs=pl.BlockSpec((RB, D), lambda i: (i, 0)),
        out_shape=jax.ShapeDtypeStruct((N2, D), jnp.float32),
    )(x, W, dinvb)


def _comb_mm_body(p_ref, dinv_ref, b_ref, w_ref, h_ref):
    t = p_ref[...] * dinv_ref[...] + b_ref[...]
    h = jnp.dot(t, w_ref[...], preferred_element_type=jnp.float32)
    h_ref[...] = h * dinv_ref[...]


def _comb_mm_call(p, dinvb, b2, W):
    return pl.pallas_call(
        _comb_mm_body,
        grid=(NBLK,),
        in_specs=[
            pl.BlockSpec((RB, D), lambda i: (i, 0)),
            pl.BlockSpec((RB, D), lambda i: (i, 0)),
            pl.BlockSpec((1, D), lambda i: (0, 0)),
            pl.BlockSpec((D, D), lambda i: (0, 0)),
        ],
        out_specs=pl.BlockSpec((RB, D), lambda i: (i, 0)),
        out_shape=jax.ShapeDtypeStruct((N2, D), jnp.float32),
    )(p, dinvb, b2, W)


def _pool(t, freq_row):
    kept = (freq_row >= 0.5).astype(jnp.float32)
    seg = lax.broadcasted_iota(jnp.int32, (16, PRB), 1) // PER
    gid = lax.broadcasted_iota(jnp.int32, (16, PRB), 0)
    K = jnp.where(seg == gid, 1.0, 0.0) * kept
    ssum = jnp.dot(K, t, preferred_element_type=jnp.float32)
    cnt = jnp.sum(K, axis=1, keepdims=True)
    return ssum / jnp.maximum(cnt, 1.0)


def _final_body(p_ref, dinv_ref, b_ref, freq_ref, pool_ref):
    t = p_ref[...] * dinv_ref[...] + b_ref[...]
    fr = freq_ref[...].reshape(1, PRB)
    pool_ref[...] = _pool(t, fr).reshape(1, 1, 16, D)


def _final_call(p, dinvb, b2, freq3):
    return pl.pallas_call(
        _final_body,
        grid=(5,),
        in_specs=[
            pl.BlockSpec((PRB, D), lambda i: (i, 0)),
            pl.BlockSpec((PRB, D), lambda i: (i, 0)),
            pl.BlockSpec((1, D), lambda i: (0, 0)),
            pl.BlockSpec((1, 1, PRB), lambda i: (i, 0, 0)),
        ],
        out_specs=pl.BlockSpec((1, 1, 16, D), lambda i: (i, 0, 0, 0)),
        out_shape=jax.ShapeDtypeStruct((5, 1, 16, D), jnp.float32),
    )(p, dinvb, b2, freq3)


def _final_center_body(p_ref, dinv_ref, b_ref, freq_ref, ctr_ref,
                       pool_ref, ctro_ref):
    t = p_ref[...] * dinv_ref[...] + b_ref[...]
    fr = freq_ref[...].reshape(1, PRB)
    pool_ref[...] = _pool(t, fr).reshape(1, 1, 16, D)
    idxl = ctr_ref[...].reshape(1, GPB * 20)
    rowid = lax.broadcasted_iota(jnp.int32, (PRB, GPB * 20), 0)
    selT = (rowid == idxl).astype(jnp.float32)
    ctr = lax.dot_general(selT, t, (((0,), (0,)), ((), ())),
                          preferred_element_type=jnp.float32)
    ctro_ref[...] = ctr.reshape(1, 1, GPB * 20, D)


def _final_center_call(p, dinvb, b2, freq3, ctr3):
    return pl.pallas_call(
        _final_center_body,
        grid=(5,),
        in_specs=[
            pl.BlockSpec((PRB, D), lambda i: (i, 0)),
            pl.BlockSpec((PRB, D), lambda i: (i, 0)),
            pl.BlockSpec((1, D), lambda i: (0, 0)),
            pl.BlockSpec((1, 1, PRB), lambda i: (i, 0, 0)),
            pl.BlockSpec((1, 1, GPB * 20), lambda i: (i, 0, 0)),
        ],
        out_specs=[
            pl.BlockSpec((1, 1, 16, D), lambda i: (i, 0, 0, 0)),
            pl.BlockSpec((1, 1, GPB * 20, D), lambda i: (i, 0, 0, 0)),
        ],
        out_shape=[
            jax.ShapeDtypeStruct((5, 1, 16, D), jnp.float32),
            jax.ShapeDtypeStruct((5, 1, GPB * 20, D), jnp.float32),
        ],
    )(p, dinvb, b2, freq3, ctr3)


def _attn_body(tp_ref, gp_ref, wv_ref, bv_ref, wo_ref, bo_ref,
               wl_ref, bl_ref, out_ref):
    per = jnp.concatenate([tp_ref[...], gp_ref[...]], axis=1)
    nt = (((1,), (1,)), ((), ()))
    v = lax.dot_general(per, wv_ref[...], nt,
                        preferred_element_type=jnp.float32) + bv_ref[...]
    o = lax.dot_general(v, wo_ref[...], nt,
                        preferred_element_type=jnp.float32) + bo_ref[...]
    up = lax.dot_general(o, wl_ref[...], nt,
                         preferred_element_type=jnp.float32) + bl_ref[...]
    out_ref[...] = up


def _attn_call(tp, gp, Wv, bv, Wo, bo, Wl, bl):
    e2 = 2 * D
    full = lambda shape: pl.BlockSpec(shape, lambda: tuple(0 for _ in shape))
    return pl.pallas_call(
        _attn_body,
        in_specs=[
            full((B, D)), full((B, D)),
            full((e2, e2)), full((1, e2)),
            full((e2, e2)), full((1, e2)),
            full((D, e2)), full((1, D)),
        ],
        out_specs=full((B, D)),
        out_shape=jax.ShapeDtypeStruct((B, D), jnp.float32),
    )(tp, gp, Wv, bv, Wo, bo, Wl, bl)


def kernel(center_traj, traj_x, traj_edge_index, traj_freq, geo_x,
           geo_edge_index, geo_freq, ptr, batch, traj_W, traj_b, geo_W,
           geo_b, Wqkv, bqkv, Wo, bo, Wl, bl):
    pad = (jnp.arange((NROWP - NROW) * CH, dtype=jnp.int32) % NPAD + N
           ).reshape(NROWP - NROW, CH)

    def _rows(e):
        return jnp.concatenate([e.reshape(NROW, CH), pad], axis=0)

    srcT = _rows(traj_edge_index[0])
    dstT = _rows(traj_edge_index[1])
    srcG = _rows(geo_edge_index[0])
    dstG = _rows(geo_edge_index[1])
    ones16 = jnp.ones((N2, 16), jnp.float32)
    x2T = jnp.pad(traj_x, ((0, NPAD), (0, 0)))
    x2G = jnp.pad(geo_x, ((0, NPAD), (0, 0)))

    degT, degG = _deg_call(dstT, dstG, ones16)
    dinvT, dinvG = _dinv_call(degT, degG)

    hT = _mm_call(x2T, traj_W[0], dinvT)
    hG = _mm_call(x2G, geo_W[0], dinvG)
    pT, pG = _edge_call(hT, hG, srcT, dstT, srcG, dstG)
    hT = _comb_mm_call(pT, dinvT, traj_b[0:1], traj_W[1])
    hG = _comb_mm_call(pG, dinvG, geo_b[0:1], geo_W[1])
    pT, pG = _edge_call(hT, hG, srcT, dstT, srcG, dstG)
    hT = _comb_mm_call(pT, dinvT, traj_b[1:2], traj_W[2])
    hG = _comb_mm_call(pG, dinvG, geo_b[1:2], geo_W[2])
    pT, pG = _edge_call(hT, hG, srcT, dstT, srcG, dstG)

    freqT3 = traj_freq.reshape(5, 1, PRB)
    freqG3 = geo_freq.reshape(5, 1, PRB)
    gidx = jnp.arange(B, dtype=jnp.int32)
    local = center_traj + (ptr[:-1] - (gidx // GPB) * PRB)[:, None]
    ctr3 = local.reshape(5, 1, GPB * 20).astype(jnp.int32)

    poolT, ctrE = _final_center_call(pT, dinvT, traj_b[2:3], freqT3, ctr3)
    poolG = _final_call(pG, dinvG, geo_b[2:3], freqG3)

    tp = poolT.reshape(5, 16, D)[:, :GPB].reshape(B, D)
    gp = poolG.reshape(5, 16, D)[:, :GPB].reshape(B, D)

    e2 = 2 * D
    Wv = Wqkv[2 * e2:]
    bv = bqkv[2 * e2:].reshape(1, e2)
    up = _attn_call(tp, gp, Wv, bv, Wo, bo.reshape(1, e2),
                    Wl, bl.reshape(1, D))

    center_traj_emb = ctrE.reshape(B, 20, D)
    user_perfence = up.reshape(B, 1, D)
    return (center_traj_emb, user_perfence)
